# baseline (speedup 1.0000x reference)
import functools

import jax
import jax.numpy as jnp
from jax import lax
from jax.experimental import pallas as pl
from jax.experimental.pallas import tpu as pltpu
from jax.experimental.pallas import tpu_sc as plsc

BATCH = 8
SEQ = 4096
DIM = 64
N_HASHES = 8
BUCKET_SIZE = 64
N_BUCKETS = SEQ // BUCKET_SIZE
N_CHUNKS = SEQ // BUCKET_SIZE
TASKS = BATCH * N_HASHES
NEG_SELF = -10000.0
GCHUNK = 128
NG = 2
GB = BATCH // NG
GT = TASKS // NG


def _hash_sort_body(qk_ref, rot_ref, pos_ref, off_ref, cnt_ref):
    qk = qk_ref[0]
    rot = rot_ref[0]
    r = lax.dot_general(qk, rot, (((1,), (0,)), ((), ())),
                        preferred_element_type=jnp.float32)
    r2 = jnp.concatenate([r, -r], axis=-1)
    m = jnp.max(r2, axis=-1, keepdims=True)
    col = lax.broadcasted_iota(jnp.int32, r2.shape, 1)
    bucket = jnp.min(jnp.where(r2 == m, col, N_BUCKETS), axis=-1,
                     keepdims=True)

    onehot = (bucket == lax.broadcasted_iota(
        jnp.int32, (SEQ, N_BUCKETS), 1)).astype(jnp.float32)

    cum = onehot
    k = 1
    while k < SEQ:
        cum = cum + jnp.concatenate(
            [jnp.zeros((k, N_BUCKETS), jnp.float32), cum[:-k]], axis=0)
        k *= 2

    counts = jnp.sum(onehot, axis=0, keepdims=True)
    ci = counts.astype(jnp.int32)
    hi = (ci >> 8).astype(jnp.float32)
    lo = (ci & 255).astype(jnp.float32)
    tri = (lax.broadcasted_iota(jnp.int32, (N_BUCKETS, N_BUCKETS), 0)
           < lax.broadcasted_iota(
               jnp.int32, (N_BUCKETS, N_BUCKETS), 1)).astype(jnp.float32)
    off = (lax.dot_general(hi, tri, (((1,), (0,)), ((), ()))) * 256.0
           + lax.dot_general(lo, tri, (((1,), (0,)), ((), ()))))

    pos = jnp.sum(onehot * (cum - 1.0 + off), axis=-1, keepdims=True)
    pos_ref[0] = pos.astype(jnp.int32).reshape(SEQ // 128, 128)
    off_ref[0] = off.astype(jnp.int32)
    cnt_ref[0] = ci


def _hash_sort(qk, rot_t):
    return pl.pallas_call(
        _hash_sort_body,
        grid=(GB, N_HASHES),
        in_specs=[
            pl.BlockSpec((1, SEQ, DIM), lambda b, h: (b, 0, 0)),
            pl.BlockSpec((1, DIM, N_BUCKETS // 2), lambda b, h: (h, 0, 0)),
        ],
        out_specs=[
            pl.BlockSpec((1, SEQ // 128, 128),
                         lambda b, h: (b * N_HASHES + h, 0, 0)),
            pl.BlockSpec((1, 1, N_BUCKETS),
                         lambda b, h: (b * N_HASHES + h, 0, 0)),
            pl.BlockSpec((1, 1, N_BUCKETS),
                         lambda b, h: (b * N_HASHES + h, 0, 0)),
        ],
        out_shape=[
            jax.ShapeDtypeStruct((GT, SEQ // 128, 128), jnp.int32),
            jax.ShapeDtypeStruct((GT, 1, N_BUCKETS), jnp.int32),
            jax.ShapeDtypeStruct((GT, 1, N_BUCKETS), jnp.int32),
        ],
    )(qk, rot_t)


def _permute_body(g, qk_hbm, v_hbm, pos_hbm, sqk_hbm, sv_hbm,
                  pos_v, idxg_v, bufq, bufv, sem):
    nc = 2
    wid = lax.axis_index("s") * nc + lax.axis_index("c")
    tl = wid // 2
    half = wid % 2
    b = GB * g + tl // N_HASHES
    toff = pl.multiple_of(tl * SEQ, SEQ)
    pltpu.sync_copy(pos_hbm.at[pl.ds(toff, SEQ)], pos_v)

    boff = b * SEQ

    def scat(j, _):
        base = pl.multiple_of(j * 16, 16)
        idx = pos_v[pl.ds(base, 16)]
        tok = lax.iota(jnp.int32, 16) + base
        plsc.store_scatter(idxg_v, [idx], tok + boff)
        return 0

    lax.fori_loop(0, SEQ // 16, scat, 0)

    def gat(gg, _):
        goff = pl.multiple_of((half * 4 + gg) * (4 * GCHUNK), 4 * GCHUNK)
        waits = []
        for j in range(4):
            idxs = idxg_v.at[pl.ds(goff + j * GCHUNK, GCHUNK)]
            dq = bufq.at[pl.ds(j * GCHUNK, GCHUNK)]
            dv = bufv.at[pl.ds(j * GCHUNK, GCHUNK)]
            waits.append(pltpu.async_copy(qk_hbm.at[idxs], dq, sem))
            waits.append(pltpu.async_copy(v_hbm.at[idxs], dv, sem))
        for w in waits:
            w.wait()
        pltpu.sync_copy(bufq, sqk_hbm.at[pl.ds(toff + goff, 4 * GCHUNK)])
        pltpu.sync_copy(bufv, sv_hbm.at[pl.ds(toff + goff, 4 * GCHUNK)])
        return 0

    lax.fori_loop(0, SEQ // (8 * GCHUNK), gat, 0)


def _permute(qk_flat, v_flat, pos_flat, g):
    tot = GT * SEQ
    mesh = plsc.VectorSubcoreMesh(core_axis_name="c", subcore_axis_name="s")
    fn = functools.partial(
        pl.kernel,
        out_type=[
            jax.ShapeDtypeStruct((tot, DIM), jnp.float32),
            jax.ShapeDtypeStruct((tot, DIM), jnp.float32),
        ],
        mesh=mesh,
        compiler_params=pltpu.CompilerParams(
            needs_layout_passes=False, use_tc_tiling_on_sc=False),
        scratch_types=[
            pltpu.VMEM((SEQ,), jnp.int32),
            pltpu.VMEM((SEQ,), jnp.int32),
            pltpu.VMEM((4 * GCHUNK, DIM), jnp.float32),
            pltpu.VMEM((4 * GCHUNK, DIM), jnp.float32),
            pltpu.SemaphoreType.DMA,
        ],
    )(functools.partial(_permute_body, g))
    return fn(qk_flat, v_flat, pos_flat)


def _attend_body(sqk_ref, sv_ref, off_ref, cnt_ref, so_ref, slse_ref):
    sqk = sqk_ref[0]
    sv = sv_ref[0]
    off = off_ref[0]
    cnt = cnt_ref[0]

    norms = jnp.sqrt(jnp.sum(sqk * sqk, axis=-1, keepdims=True))
    kn = sqk / jnp.maximum(norms, 1e-12)

    q = sqk.reshape(N_CHUNKS, BUCKET_SIZE, DIM)
    kc = kn.reshape(N_CHUNKS, BUCKET_SIZE, DIM)
    kwin = jnp.concatenate(
        [kc, jnp.concatenate([kc[-1:], kc[:-1]], axis=0)], axis=1)
    vc = sv.reshape(N_CHUNKS, BUCKET_SIZE, DIM)
    vwin = jnp.concatenate(
        [vc, jnp.concatenate([vc[-1:], vc[:-1]], axis=0)], axis=1)

    ji = lax.broadcasted_iota(jnp.int32, (SEQ, N_BUCKETS), 0)
    ohs = ((ji >= off) & (ji < off + cnt)).astype(jnp.float32)
    oq = ohs.reshape(N_CHUNKS, BUCKET_SIZE, N_BUCKETS)
    okw = jnp.concatenate(
        [oq, jnp.concatenate(
            [jnp.zeros((1, BUCKET_SIZE, N_BUCKETS), jnp.float32), oq[:-1]],
            axis=0)], axis=1)
    same = lax.dot_general(oq, okw, (((2,), (2,)), ((0,), (0,))))

    dots = lax.dot_general(q, kwin, (((2,), (2,)), ((0,), (0,))))
    dots = dots * (DIM ** -0.5)
    qi = lax.broadcasted_iota(jnp.int32, dots.shape, 1)
    zi = lax.broadcasted_iota(jnp.int32, dots.shape, 2)
    dots = jnp.where(qi == zi, NEG_SELF, dots)
    dots = jnp.where(same < 0.5, -jnp.finfo(jnp.float32).max, dots)

    m = jnp.max(dots, axis=-1, keepdims=True)
    e = jnp.exp(dots - m)
    s = jnp.sum(e, axis=-1, keepdims=True)
    lse = m + jnp.log(s)
    p = e * (1.0 / s)
    bo = lax.dot_general(p, vwin, (((2,), (1,)), ((0,), (0,))))
    so_ref[0] = bo.reshape(SEQ, DIM)
    slse_ref[0] = lse.reshape(SEQ, 1).reshape(SEQ // 128, 128)


def _attend(sqk, sv, offs, cnts):
    return pl.pallas_call(
        _attend_body,
        grid=(GT,),
        in_specs=[
            pl.BlockSpec((1, SEQ, DIM), lambda t: (t, 0, 0)),
            pl.BlockSpec((1, SEQ, DIM), lambda t: (t, 0, 0)),
            pl.BlockSpec((1, 1, N_BUCKETS), lambda t: (t, 0, 0)),
            pl.BlockSpec((1, 1, N_BUCKETS), lambda t: (t, 0, 0)),
        ],
        out_specs=[
            pl.BlockSpec((1, SEQ, DIM), lambda t: (t, 0, 0)),
            pl.BlockSpec((1, SEQ // 128, 128), lambda t: (t, 0, 0)),
        ],
        out_shape=[
            jax.ShapeDtypeStruct((GT, SEQ, DIM), jnp.float32),
            jax.ShapeDtypeStruct((GT, SEQ // 128, 128), jnp.float32),
        ],
    )(sqk, sv, offs, cnts)


def _unpermute_body(so_hbm, slse_hbm, pos_hbm, o_hbm, lg_hbm,
                    pos_v, gidx_v, lse_v, lgo_v, bufo, sem):
    nc = 2
    wid = lax.axis_index("s") * nc + lax.axis_index("c")
    tl = wid // 2
    half = wid % 2
    toff = pl.multiple_of(tl * SEQ, SEQ)
    pltpu.sync_copy(pos_hbm.at[pl.ds(toff, SEQ)], pos_v)
    pltpu.sync_copy(slse_hbm.at[pl.ds(toff, SEQ)], lse_v)

    hoff = half * (SEQ // 2)

    def addoff(j, _):
        base = pl.multiple_of(hoff + j * 16, 16)
        idx = pos_v[pl.ds(base, 16)]
        gidx_v[pl.ds(base, 16)] = idx + toff
        lgo_v[pl.ds(base, 16)] = plsc.load_gather(lse_v, [idx])
        return 0

    lax.fori_loop(0, SEQ // 32, addoff, 0)
    pltpu.sync_copy(lgo_v.at[pl.ds(hoff, SEQ // 2)],
                    lg_hbm.at[pl.ds(toff + hoff, SEQ // 2)])

    def gat(gg, _):
        goff = pl.multiple_of((half * 4 + gg) * (4 * GCHUNK), 4 * GCHUNK)
        waits = []
        for j in range(4):
            idxs = gidx_v.at[pl.ds(goff + j * GCHUNK, GCHUNK)]
            do = bufo.at[pl.ds(j * GCHUNK, GCHUNK)]
            waits.append(pltpu.async_copy(so_hbm.at[idxs], do, sem))
        for w in waits:
            w.wait()
        pltpu.sync_copy(bufo, o_hbm.at[pl.ds(toff + goff, 4 * GCHUNK)])
        return 0

    lax.fori_loop(0, SEQ // (8 * GCHUNK), gat, 0)


def _unpermute(so_flat, slse_flat, pos_flat):
    tot = GT * SEQ
    mesh = plsc.VectorSubcoreMesh(core_axis_name="c", subcore_axis_name="s")
    fn = functools.partial(
        pl.kernel,
        out_type=[
            jax.ShapeDtypeStruct((tot, DIM), jnp.float32),
            jax.ShapeDtypeStruct((tot,), jnp.float32),
        ],
        mesh=mesh,
        compiler_params=pltpu.CompilerParams(
            needs_layout_passes=False, use_tc_tiling_on_sc=False),
        scratch_types=[
            pltpu.VMEM((SEQ,), jnp.int32),
            pltpu.VMEM((SEQ,), jnp.int32),
            pltpu.VMEM((SEQ,), jnp.float32),
            pltpu.VMEM((SEQ,), jnp.float32),
            pltpu.VMEM((4 * GCHUNK, DIM), jnp.float32),
            pltpu.SemaphoreType.DMA,
        ],
    )(_unpermute_body)
    return fn(so_flat, slse_flat, pos_flat)


_CSEQ = 512


def _combine_body(o_ref, lg_ref, out_ref):
    o = o_ref[0]
    lg = lg_ref[0]
    m = jnp.max(lg, axis=-1, keepdims=True)
    e = jnp.exp(lg - m)
    s = jnp.sum(e, axis=-1, keepdims=True)
    p = e / s
    acc = o[0] * p[:, 0:1]
    for h in range(1, N_HASHES):
        acc = acc + o[h] * p[:, h:h + 1]
    out_ref[0] = acc


def _combine(o4, lg3t):
    return pl.pallas_call(
        _combine_body,
        grid=(GB, SEQ // _CSEQ),
        in_specs=[
            pl.BlockSpec((1, N_HASHES, _CSEQ, DIM), lambda b, s: (b, 0, s, 0)),
            pl.BlockSpec((1, _CSEQ, N_HASHES), lambda b, s: (b, s, 0)),
        ],
        out_specs=pl.BlockSpec((1, _CSEQ, DIM), lambda b, s: (b, s, 0)),
        out_shape=jax.ShapeDtypeStruct((GB, SEQ, DIM), jnp.float32),
    )(o4, lg3t)


def kernel(qk, v):
    rot = jax.random.normal(jax.random.key(42),
                            (DIM, N_HASHES, N_BUCKETS // 2), dtype=qk.dtype)
    rot_t = jnp.transpose(rot, (1, 0, 2))

    qk_flat = qk.reshape(BATCH * SEQ, DIM)
    v_flat = v.reshape(BATCH * SEQ, DIM)

    outs = []
    for g in range(NG):
        pos, offs, cnts = _hash_sort(qk[g * GB:(g + 1) * GB], rot_t)
        pos_flat = pos.reshape(GT * SEQ)
        sqk_flat, sv_flat = _permute(qk_flat, v_flat, pos_flat, g)
        so, slse = _attend(sqk_flat.reshape(GT, SEQ, DIM),
                           sv_flat.reshape(GT, SEQ, DIM),
                           offs, cnts)
        o_flat, lg_flat = _unpermute(so.reshape(GT * SEQ, DIM),
                                     slse.reshape(GT * SEQ),
                                     pos_flat)
        lg3t = jnp.transpose(lg_flat.reshape(GB, N_HASHES, SEQ), (0, 2, 1))
        outs.append(_combine(o_flat.reshape(GB, N_HASHES, SEQ, DIM), lg3t))
    return jnp.concatenate(outs, axis=0)
